# baseline (speedup 1.0000x reference)
import functools

import jax
import jax.numpy as jnp
from jax import lax
from jax.experimental import pallas as pl
from jax.experimental.pallas import tpu as pltpu
from jax.experimental.pallas import tpu_sc as plsc

N = 10000
E = 320000
D = 128

CHUNK = 128
NC, NS = 2, 16
NW = NC * NS
CPT = 80
HCPT = CPT // 2
EP = NW * CPT * CHUNK
ZCHUNK = 80
NZ = N // ZCHUNK
ZITERS = (NZ + NS - 1) // NS


def _mm_kernel(x_ref, w_ref, b_ref, o_ref):
    o_ref[...] = (
        jnp.dot(x_ref[...], w_ref[...], preferred_element_type=jnp.float32)
        + b_ref[...]
    )


def _matmul(x, W, b):
    BN = 2000
    return pl.pallas_call(
        _mm_kernel,
        grid=(N // BN,),
        in_specs=[
            pl.BlockSpec((BN, D), lambda i: (i, 0)),
            pl.BlockSpec((D, D), lambda i: (0, 0)),
            pl.BlockSpec((1, D), lambda i: (0, 0)),
        ],
        out_specs=pl.BlockSpec((BN, D), lambda i: (i, 0)),
        out_shape=jax.ShapeDtypeStruct((N, D), jnp.float32),
    )(x, W, b.reshape(1, D))


def _ep_kernel(p_ref, x_ref, o_ref):
    o_ref[...] = jnp.maximum(p_ref[0] + p_ref[1], 0.0) + x_ref[...]


def _epilogue(partial, x):
    BN = 2000
    return pl.pallas_call(
        _ep_kernel,
        grid=(N // BN,),
        in_specs=[
            pl.BlockSpec((2, BN, D), lambda i: (0, i, 0)),
            pl.BlockSpec((BN, D), lambda i: (i, 0)),
        ],
        out_specs=pl.BlockSpec((BN, D), lambda i: (i, 0)),
        out_shape=jax.ShapeDtypeStruct((N, D), jnp.float32),
    )(partial, x)


def _scale_rows(rows, w_all, i):
    def scale(g, c2):
        w16 = w_all[i, pl.ds(g * 16, 16)]
        for k in range(16):
            s = w16[k]
            r = g * 16 + k
            for j in range(D // 16):
                sl = pl.ds(16 * j, 16)
                rows[r, sl] = rows[r, sl] * s
        return c2

    lax.fori_loop(0, CHUNK // 16, scale, 0)


def _edge_body(hidden_hbm, src_hbm, dst_hbm, w_hbm, partial_hbm,
               src_all, dst_all, w_all, rows0, rows1, acc, sem0, sem1):
    cid = lax.axis_index("c")
    sid = lax.axis_index("s")
    wid = sid * NC + cid

    zero = jnp.zeros((16,), jnp.float32)

    def zrow(r, carry):
        for j in range(D // 16):
            rows0[r, pl.ds(16 * j, 16)] = zero
        return carry

    lax.fori_loop(0, ZCHUNK, zrow, 0)
    for k in range(ZITERS):
        c = sid + NS * k

        @pl.when(c < NZ)
        def _():
            pltpu.sync_copy(
                rows0.at[pl.ds(0, ZCHUNK)],
                acc.at[pl.ds(c * ZCHUNK, ZCHUNK)],
            )

    plsc.subcore_barrier()

    for h in range(CPT // HCPT):
        base = wid * CPT + h * HCPT
        pltpu.sync_copy(src_hbm.at[pl.ds(base, HCPT)], src_all)
        pltpu.sync_copy(dst_hbm.at[pl.ds(base, HCPT)], dst_all)
        pltpu.sync_copy(w_hbm.at[pl.ds(base, HCPT)], w_all)

        pltpu.make_async_copy(hidden_hbm.at[src_all.at[0]], rows0, sem0).start()

        def body(g, carry):
            i0 = 2 * g
            i1 = 2 * g + 1

            pltpu.make_async_copy(
                hidden_hbm.at[src_all.at[i0]], rows0, sem0).wait()
            pltpu.make_async_copy(hidden_hbm.at[src_all.at[i1]], rows1, sem1).start()
            _scale_rows(rows0, w_all, i0)
            pltpu.sync_copy(rows0, acc.at[dst_all.at[i0]], add=True)

            pltpu.make_async_copy(
                hidden_hbm.at[src_all.at[i1]], rows1, sem1).wait()

            @pl.when(i0 + 2 < HCPT)
            def _():
                pltpu.make_async_copy(
                    hidden_hbm.at[src_all.at[i0 + 2]], rows0, sem0).start()

            _scale_rows(rows1, w_all, i1)
            pltpu.sync_copy(rows1, acc.at[dst_all.at[i1]], add=True)
            return carry

        lax.fori_loop(0, HCPT // 2, body, 0)

    plsc.subcore_barrier()

    for k in range(ZITERS):
        c = sid + NS * k

        @pl.when(c < NZ)
        def _():
            pltpu.sync_copy(
                acc.at[pl.ds(c * ZCHUNK, ZCHUNK)],
                partial_hbm.at[cid, pl.ds(c * ZCHUNK, ZCHUNK)],
            )


def _edge_pass(hidden, src2d, dst2d, w2d):
    mesh = plsc.VectorSubcoreMesh(core_axis_name="c", subcore_axis_name="s")
    f = functools.partial(
        pl.kernel,
        mesh=mesh,
        out_type=jax.ShapeDtypeStruct((NC, N, D), jnp.float32),
        scratch_types=[
            pltpu.VMEM((HCPT, CHUNK), jnp.int32),
            pltpu.VMEM((HCPT, CHUNK), jnp.int32),
            pltpu.VMEM((HCPT, CHUNK), jnp.float32),
            pltpu.VMEM((CHUNK, D), jnp.float32),
            pltpu.VMEM((CHUNK, D), jnp.float32),
            pltpu.VMEM_SHARED((N, D), jnp.float32),
            pltpu.SemaphoreType.DMA,
            pltpu.SemaphoreType.DMA,
        ],
    )(_edge_body)
    return f(hidden, src2d, dst2d, w2d)


def kernel(x, edge_index, edge_weight, W, b):
    hidden = _matmul(x, W, b)
    pad = EP - E
    src2d = jnp.concatenate(
        [edge_index[0], jnp.zeros((pad,), jnp.int32)]).reshape(EP // CHUNK, CHUNK)
    dst2d = jnp.concatenate(
        [edge_index[1], jnp.zeros((pad,), jnp.int32)]).reshape(EP // CHUNK, CHUNK)
    w2d = jnp.concatenate(
        [edge_weight, jnp.zeros((pad,), jnp.float32)]).reshape(EP // CHUNK, CHUNK)
    partial = _edge_pass(hidden, src2d, dst2d, w2d)
    return _epilogue(partial, x)
